# Initial kernel scaffold; baseline (speedup 1.0000x reference)
import jax
import jax.numpy as jnp
from jax.experimental import pallas as pl


def kernel(x):
    raise NotImplementedError("write your pallas kernel here")


# speedup vs baseline: 37.3782x; 37.3782x over previous
import functools

import jax
import jax.numpy as jnp
from jax.experimental import pallas as pl
from jax.experimental.pallas import tpu as pltpu

_K = 256
_BLOCK_ROWS = 64


def _topk_mask_kernel(x_ref, o_ref, *, k):
    x = x_ref[...]
    i32 = jax.lax.bitcast_convert_type(x, jnp.int32)
    key = i32 ^ (jax.lax.shift_right_arithmetic(i32, 31) & jnp.int32(0x7FFFFFFF))

    cnt = jnp.sum((key >= 0).astype(jnp.int32), axis=1, keepdims=True)
    prefix = jnp.where(cnt >= k, jnp.int32(0), jnp.int32(-2147483648))
    for bit in range(30, -1, -1):
        cand = prefix | jnp.int32(1 << bit)
        cnt = jnp.sum((key >= cand).astype(jnp.int32), axis=1, keepdims=True)
        prefix = jnp.where(cnt >= k, cand, prefix)

    thresh = jnp.maximum(prefix, jnp.int32(1))
    o_ref[...] = jnp.where(key >= thresh, x, jnp.float32(0.0))


def kernel(x):
    rows, cols = x.shape
    br = min(_BLOCK_ROWS, rows)
    return pl.pallas_call(
        functools.partial(_topk_mask_kernel, k=_K),
        grid=(rows // br,),
        in_specs=[pl.BlockSpec((br, cols), lambda i: (i, 0))],
        out_specs=pl.BlockSpec((br, cols), lambda i: (i, 0)),
        out_shape=jax.ShapeDtypeStruct(x.shape, x.dtype),
        compiler_params=pltpu.CompilerParams(
            dimension_semantics=("arbitrary",)),
    )(x)
